# baseline (device time: 263861 ns/iter reference)
import jax
import jax.numpy as jnp
from jax import lax
from jax.experimental import pallas as pl
from jax.experimental.pallas import tpu as pltpu

N_DEV = 16


def _ring_allreduce(x2d, collective_id):
    rows, cols = x2d.shape
    assert rows % N_DEV == 0
    chunk = rows // N_DEV
    n_steps = N_DEV - 1

    def body(x_ref, out_ref, comm_ref, rs_send, rs_recv, ag_send, ag_recv):
        my = lax.axis_index("i")
        left = (my + N_DEV - 1) % N_DEV
        right = (my + 1) % N_DEV

        barrier_sem = pltpu.get_barrier_semaphore()
        for nbr in (left, right):
            pl.semaphore_signal(
                barrier_sem, inc=1,
                device_id=(nbr,), device_id_type=pl.DeviceIdType.MESH,
            )
        pl.semaphore_wait(barrier_sem, 2)

        out_ref[...] = x_ref[...]

        for s in range(n_steps):
            c_send = (my - s) % N_DEV
            rdma = pltpu.make_async_remote_copy(
                src_ref=out_ref.at[pl.ds(c_send * chunk, chunk)],
                dst_ref=comm_ref.at[s],
                send_sem=rs_send.at[s],
                recv_sem=rs_recv.at[s],
                device_id=(right,),
                device_id_type=pl.DeviceIdType.MESH,
            )
            rdma.start()
            rdma.wait()
            c_acc = (my - s - 1) % N_DEV
            out_ref[pl.ds(c_acc * chunk, chunk), :] += comm_ref[s]

        for s in range(n_steps):
            c = (my + 1 - s) % N_DEV
            rdma = pltpu.make_async_remote_copy(
                src_ref=out_ref.at[pl.ds(c * chunk, chunk)],
                dst_ref=out_ref.at[pl.ds(c * chunk, chunk)],
                send_sem=ag_send.at[s],
                recv_sem=ag_recv.at[s],
                device_id=(right,),
                device_id_type=pl.DeviceIdType.MESH,
            )
            rdma.start()
            rdma.wait()

    return pl.pallas_call(
        body,
        out_shape=jax.ShapeDtypeStruct((rows, cols), x2d.dtype),
        in_specs=[pl.BlockSpec(memory_space=pltpu.VMEM)],
        out_specs=pl.BlockSpec(memory_space=pltpu.VMEM),
        scratch_shapes=[
            pltpu.VMEM((n_steps, chunk, cols), x2d.dtype),
            pltpu.SemaphoreType.DMA((n_steps,)),
            pltpu.SemaphoreType.DMA((n_steps,)),
            pltpu.SemaphoreType.DMA((n_steps,)),
            pltpu.SemaphoreType.DMA((n_steps,)),
        ],
        compiler_params=pltpu.CompilerParams(collective_id=collective_id),
    )(x2d)


def kernel(x, Wq, Wk, Wv, Wo, t_emb, W_mod, W_ff1, W_ff2):
    B, S, D = x.shape
    eps = 1e-5
    Dh = 96

    mod = t_emb @ W_mod
    sa, sha, ga, sm, shm, gm = jnp.split(mod, 6, axis=-1)

    def ln(h):
        m = h.mean(axis=-1, keepdims=True)
        v = h.var(axis=-1, keepdims=True)
        return (h - m) * lax.rsqrt(v + eps)

    x0 = x
    xa = ln(x0) * (1.0 + sa[:, None, :]) + sha[:, None, :]

    h_local = Wq.shape[1] // Dh
    Q = (xa @ Wq).reshape(B, S, h_local, Dh)
    K = (xa @ Wk).reshape(B, S, h_local, Dh)
    V = (xa @ Wv).reshape(B, S, h_local, Dh)
    scores = jnp.einsum("bihd,bjhd->bhij", Q, K) * (1.0 / (Dh ** 0.5))
    p = jax.nn.softmax(scores, axis=-1)
    o = jnp.einsum("bhij,bjhd->bihd", p, V).reshape(B, S, h_local * Dh)
    partial_attn = o @ Wo

    attn = _ring_allreduce(partial_attn.reshape(B * S, D), 0).reshape(B, S, D)
    x1 = x0 + ga[:, None, :] * attn

    xm = ln(x1) * (1.0 + sm[:, None, :]) + shm[:, None, :]
    h = xm @ W_ff1
    h = h * jax.nn.sigmoid(h)
    partial_ff = h @ W_ff2

    ff = _ring_allreduce(partial_ff.reshape(B * S, D), 1).reshape(B, S, D)
    return x1 + gm[:, None, :] * ff


# device time: 147534 ns/iter; 1.7885x vs baseline; 1.7885x over previous
import jax
import jax.numpy as jnp
from jax import lax
from jax.experimental import pallas as pl
from jax.experimental.pallas import tpu as pltpu

N_DEV = 16
PLANE = 4
GROUP = 256
HALF = 128
SUB = 32


def _ar_residual(partial2d, base2d, gate, collective_id):
    rows, cols = partial2d.shape

    def body(p_ref, base_ref, gate_ref, out_ref,
             acc, pbufA, pbufB, zbuf,
             prs_send, prs_recv, zrs_send, zrs_recv,
             zag_send, zag_recv, pag_send, pag_recv):
        my = lax.axis_index("i")
        z = my // PLANE
        k = my % PLANE
        right_p = z * PLANE + (k + 1) % PLANE
        left_p = z * PLANE + (k + 3) % PLANE
        up = ((z + 1) % PLANE) * PLANE + k
        down = ((z + 3) % PLANE) * PLANE + k

        barrier_sem = pltpu.get_barrier_semaphore()
        for nbr in (left_p, right_p, up, down):
            pl.semaphore_signal(
                barrier_sem, inc=1,
                device_id=(nbr,), device_id_type=pl.DeviceIdType.MESH,
            )
        pl.semaphore_wait(barrier_sem, 4)

        acc[...] = p_ref[...]

        for s in range(PLANE - 1):
            ga_s = (k - s) % PLANE
            gb_s = (k + s) % PLANE
            cw = pltpu.make_async_remote_copy(
                src_ref=acc.at[pl.ds(ga_s * GROUP, HALF)],
                dst_ref=pbufA.at[s],
                send_sem=prs_send.at[s, 0],
                recv_sem=prs_recv.at[s, 0],
                device_id=(right_p,),
                device_id_type=pl.DeviceIdType.MESH,
            )
            ccw = pltpu.make_async_remote_copy(
                src_ref=acc.at[pl.ds(gb_s * GROUP + HALF, HALF)],
                dst_ref=pbufB.at[s],
                send_sem=prs_send.at[s, 1],
                recv_sem=prs_recv.at[s, 1],
                device_id=(left_p,),
                device_id_type=pl.DeviceIdType.MESH,
            )
            cw.start()
            ccw.start()
            cw.wait()
            ccw.wait()
            ga_r = (k - s - 1) % PLANE
            gb_r = (k + s + 1) % PLANE
            acc[pl.ds(ga_r * GROUP, HALF), :] += pbufA[s]
            acc[pl.ds(gb_r * GROUP + HALF, HALF), :] += pbufB[s]

        A_k = (k + 1) % PLANE
        B_k = (k + 3) % PLANE
        ra = A_k * GROUP
        rb = B_k * GROUP + HALF

        for s in range(PLANE - 1):
            j_s = (z - s) % PLANE
            za = pltpu.make_async_remote_copy(
                src_ref=acc.at[pl.ds(ra + j_s * SUB, SUB)],
                dst_ref=zbuf.at[s, 0],
                send_sem=zrs_send.at[s, 0],
                recv_sem=zrs_recv.at[s, 0],
                device_id=(up,),
                device_id_type=pl.DeviceIdType.MESH,
            )
            zb = pltpu.make_async_remote_copy(
                src_ref=acc.at[pl.ds(rb + j_s * SUB, SUB)],
                dst_ref=zbuf.at[s, 1],
                send_sem=zrs_send.at[s, 1],
                recv_sem=zrs_recv.at[s, 1],
                device_id=(up,),
                device_id_type=pl.DeviceIdType.MESH,
            )
            za.start()
            zb.start()
            za.wait()
            zb.wait()
            j_r = (z - s - 1) % PLANE
            acc[pl.ds(ra + j_r * SUB, SUB), :] += zbuf[s, 0]
            acc[pl.ds(rb + j_r * SUB, SUB), :] += zbuf[s, 1]

        j_own = (z + 1) % PLANE
        for off in (ra + j_own * SUB, rb + j_own * SUB):
            b = off // 512
            g_row = gate_ref[pl.ds(b, 1), :]
            out_ref[pl.ds(off, SUB), :] = (
                base_ref[pl.ds(off, SUB), :] + g_row * acc[pl.ds(off, SUB), :]
            )

        for s in range(PLANE - 1):
            j_s = (z + 1 - s) % PLANE
            for t, base_row in ((0, ra), (1, rb)):
                rr = base_row + j_s * SUB
                rdma = pltpu.make_async_remote_copy(
                    src_ref=out_ref.at[pl.ds(rr, SUB)],
                    dst_ref=out_ref.at[pl.ds(rr, SUB)],
                    send_sem=zag_send.at[s, t],
                    recv_sem=zag_recv.at[s, t],
                    device_id=(up,),
                    device_id_type=pl.DeviceIdType.MESH,
                )
                if t == 0:
                    za = rdma
                else:
                    zb = rdma
            za.start()
            zb.start()
            za.wait()
            zb.wait()

        for s in range(PLANE - 1):
            ga_s = (k + 1 - s) % PLANE
            gb_s = (k + 3 + s) % PLANE
            cw = pltpu.make_async_remote_copy(
                src_ref=out_ref.at[pl.ds(ga_s * GROUP, HALF)],
                dst_ref=out_ref.at[pl.ds(ga_s * GROUP, HALF)],
                send_sem=pag_send.at[s, 0],
                recv_sem=pag_recv.at[s, 0],
                device_id=(right_p,),
                device_id_type=pl.DeviceIdType.MESH,
            )
            ccw = pltpu.make_async_remote_copy(
                src_ref=out_ref.at[pl.ds(gb_s * GROUP + HALF, HALF)],
                dst_ref=out_ref.at[pl.ds(gb_s * GROUP + HALF, HALF)],
                send_sem=pag_send.at[s, 1],
                recv_sem=pag_recv.at[s, 1],
                device_id=(left_p,),
                device_id_type=pl.DeviceIdType.MESH,
            )
            cw.start()
            ccw.start()
            cw.wait()
            ccw.wait()

    n_steps = PLANE - 1
    return pl.pallas_call(
        body,
        out_shape=jax.ShapeDtypeStruct((rows, cols), partial2d.dtype),
        in_specs=[
            pl.BlockSpec(memory_space=pltpu.VMEM),
            pl.BlockSpec(memory_space=pltpu.VMEM),
            pl.BlockSpec(memory_space=pltpu.VMEM),
        ],
        out_specs=pl.BlockSpec(memory_space=pltpu.VMEM),
        scratch_shapes=[
            pltpu.VMEM((rows, cols), partial2d.dtype),
            pltpu.VMEM((n_steps, HALF, cols), partial2d.dtype),
            pltpu.VMEM((n_steps, HALF, cols), partial2d.dtype),
            pltpu.VMEM((n_steps, 2, SUB, cols), partial2d.dtype),
            pltpu.SemaphoreType.DMA((n_steps, 2)),
            pltpu.SemaphoreType.DMA((n_steps, 2)),
            pltpu.SemaphoreType.DMA((n_steps, 2)),
            pltpu.SemaphoreType.DMA((n_steps, 2)),
            pltpu.SemaphoreType.DMA((n_steps, 2)),
            pltpu.SemaphoreType.DMA((n_steps, 2)),
            pltpu.SemaphoreType.DMA((n_steps, 2)),
            pltpu.SemaphoreType.DMA((n_steps, 2)),
        ],
        compiler_params=pltpu.CompilerParams(collective_id=collective_id),
    )(partial2d, base2d, gate)


def kernel(x, Wq, Wk, Wv, Wo, t_emb, W_mod, W_ff1, W_ff2):
    B, S, D = x.shape
    eps = 1e-5
    Dh = 96

    mod = t_emb @ W_mod
    sa, sha, ga, sm, shm, gm = jnp.split(mod, 6, axis=-1)

    def ln(h):
        m = h.mean(axis=-1, keepdims=True)
        v = h.var(axis=-1, keepdims=True)
        return (h - m) * lax.rsqrt(v + eps)

    x0 = x
    xa = ln(x0) * (1.0 + sa[:, None, :]) + sha[:, None, :]

    h_local = Wq.shape[1] // Dh
    Q = (xa @ Wq).reshape(B, S, h_local, Dh)
    K = (xa @ Wk).reshape(B, S, h_local, Dh)
    V = (xa @ Wv).reshape(B, S, h_local, Dh)
    scores = jnp.einsum("bihd,bjhd->bhij", Q, K) * (1.0 / (Dh ** 0.5))
    p = jax.nn.softmax(scores, axis=-1)
    o = jnp.einsum("bhij,bjhd->bihd", p, V).reshape(B, S, h_local * Dh)
    partial_attn = o @ Wo

    x1_2d = _ar_residual(
        partial_attn.reshape(B * S, D), x0.reshape(B * S, D), ga, 0
    )
    x1 = x1_2d.reshape(B, S, D)

    xm = ln(x1) * (1.0 + sm[:, None, :]) + shm[:, None, :]
    h = xm @ W_ff1
    h = h * jax.nn.sigmoid(h)
    partial_ff = h @ W_ff2

    out2d = _ar_residual(partial_ff.reshape(B * S, D), x1_2d, gm, 1)
    return out2d.reshape(B, S, D)


# device time: 135300 ns/iter; 1.9502x vs baseline; 1.0904x over previous
import jax
import jax.numpy as jnp
from jax import lax
from jax.experimental import pallas as pl
from jax.experimental.pallas import tpu as pltpu

N_DEV = 16
PLANE = 4
GROUP = 256
HALF = 128
SUB = 32


def _ar_residual(partial2d, base2d, gate, collective_id):
    rows, cols = partial2d.shape

    def body(p_ref, base_ref, gate_ref, out_ref,
             acc, pbufA, pbufB, zbuf,
             prs_send, prs_recv, zrs_send, zrs_recv,
             zag_send, zag_recv, cwag_send, cwag_recv,
             ccwag_send, ccwag_recv):
        my = lax.axis_index("i")
        z = my // PLANE
        k = my % PLANE
        right_p = z * PLANE + (k + 1) % PLANE
        left_p = z * PLANE + (k + 3) % PLANE
        up = ((z + 1) % PLANE) * PLANE + k
        down = ((z + 3) % PLANE) * PLANE + k

        barrier_sem = pltpu.get_barrier_semaphore()
        for nbr in (left_p, right_p, up, down):
            pl.semaphore_signal(
                barrier_sem, inc=1,
                device_id=(nbr,), device_id_type=pl.DeviceIdType.MESH,
            )
        pl.semaphore_wait(barrier_sem, 4)

        acc[...] = p_ref[...]

        for s in range(PLANE - 1):
            ga_s = (k - s) % PLANE
            gb_s = (k + s) % PLANE
            cw = pltpu.make_async_remote_copy(
                src_ref=acc.at[pl.ds(ga_s * GROUP, HALF)],
                dst_ref=pbufA.at[s],
                send_sem=prs_send.at[s, 0],
                recv_sem=prs_recv.at[s, 0],
                device_id=(right_p,),
                device_id_type=pl.DeviceIdType.MESH,
            )
            ccw = pltpu.make_async_remote_copy(
                src_ref=acc.at[pl.ds(gb_s * GROUP + HALF, HALF)],
                dst_ref=pbufB.at[s],
                send_sem=prs_send.at[s, 1],
                recv_sem=prs_recv.at[s, 1],
                device_id=(left_p,),
                device_id_type=pl.DeviceIdType.MESH,
            )
            cw.start()
            ccw.start()
            cw.wait()
            ccw.wait()
            ga_r = (k - s - 1) % PLANE
            gb_r = (k + s + 1) % PLANE
            acc[pl.ds(ga_r * GROUP, HALF), :] += pbufA[s]
            acc[pl.ds(gb_r * GROUP + HALF, HALF), :] += pbufB[s]

        A_k = (k + 1) % PLANE
        B_k = (k + 3) % PLANE
        ra = A_k * GROUP
        rb = B_k * GROUP + HALF

        for s in range(PLANE - 1):
            j_s = (z - s) % PLANE
            za = pltpu.make_async_remote_copy(
                src_ref=acc.at[pl.ds(ra + j_s * SUB, SUB)],
                dst_ref=zbuf.at[s, 0],
                send_sem=zrs_send.at[s, 0],
                recv_sem=zrs_recv.at[s, 0],
                device_id=(up,),
                device_id_type=pl.DeviceIdType.MESH,
            )
            zb = pltpu.make_async_remote_copy(
                src_ref=acc.at[pl.ds(rb + j_s * SUB, SUB)],
                dst_ref=zbuf.at[s, 1],
                send_sem=zrs_send.at[s, 1],
                recv_sem=zrs_recv.at[s, 1],
                device_id=(up,),
                device_id_type=pl.DeviceIdType.MESH,
            )
            za.start()
            zb.start()
            za.wait()
            zb.wait()
            j_r = (z - s - 1) % PLANE
            acc[pl.ds(ra + j_r * SUB, SUB), :] += zbuf[s, 0]
            acc[pl.ds(rb + j_r * SUB, SUB), :] += zbuf[s, 1]

        j_own = (z + 1) % PLANE
        for off in (ra + j_own * SUB, rb + j_own * SUB):
            b = off // 512
            g_row = gate_ref[pl.ds(b, 1), :]
            out_ref[pl.ds(off, SUB), :] = (
                base_ref[pl.ds(off, SUB), :] + g_row * acc[pl.ds(off, SUB), :]
            )

        def j_of(r):
            return (z + 1 - r) % PLANE

        def mk(rowoff, sem_send, sem_recv, dev):
            return pltpu.make_async_remote_copy(
                src_ref=out_ref.at[pl.ds(rowoff, SUB)],
                dst_ref=out_ref.at[pl.ds(rowoff, SUB)],
                send_sem=sem_send,
                recv_sem=sem_recv,
                device_id=(dev,),
                device_id_type=pl.DeviceIdType.MESH,
            )

        def cw_rows(t, r):
            return ((k + 1 - t) % PLANE) * GROUP + j_of(r) * SUB

        def ccw_rows(t, r):
            return ((k + 3 + t) % PLANE) * GROUP + HALF + j_of(r) * SUB

        started = []

        def launch(rowoff, sem_send, sem_recv, dev):
            rdma = mk(rowoff, sem_send, sem_recv, dev)
            rdma.start()
            started.append(rdma)

        launch(ra + j_of(0) * SUB, zag_send.at[0, 0], zag_recv.at[0, 0], up)
        launch(rb + j_of(0) * SUB, zag_send.at[0, 1], zag_recv.at[0, 1], up)
        launch(cw_rows(0, 0), cwag_send.at[0, 0], cwag_recv.at[0, 0], right_p)
        launch(ccw_rows(0, 0), ccwag_send.at[0, 0], ccwag_recv.at[0, 0],
               left_p)

        for s in range(1, PLANE):
            mk(ra + j_of(s) * SUB, zag_send.at[s - 1, 0],
               zag_recv.at[s - 1, 0], up).wait_recv()
            mk(rb + j_of(s) * SUB, zag_send.at[s - 1, 1],
               zag_recv.at[s - 1, 1], up).wait_recv()
            if s < PLANE - 1:
                launch(ra + j_of(s) * SUB, zag_send.at[s, 0],
                       zag_recv.at[s, 0], up)
                launch(rb + j_of(s) * SUB, zag_send.at[s, 1],
                       zag_recv.at[s, 1], up)
            launch(cw_rows(0, s), cwag_send.at[0, s], cwag_recv.at[0, s],
                   right_p)
            launch(ccw_rows(0, s), ccwag_send.at[0, s], ccwag_recv.at[0, s],
                   left_p)

        for t in (1, 2):
            for r in range(PLANE):
                mk(cw_rows(t, r), cwag_send.at[t - 1, r],
                   cwag_recv.at[t - 1, r], right_p).wait_recv()
                launch(cw_rows(t, r), cwag_send.at[t, r],
                       cwag_recv.at[t, r], right_p)
                mk(ccw_rows(t, r), ccwag_send.at[t - 1, r],
                   ccwag_recv.at[t - 1, r], left_p).wait_recv()
                launch(ccw_rows(t, r), ccwag_send.at[t, r],
                       ccwag_recv.at[t, r], left_p)

        for r in range(PLANE):
            mk(cw_rows(3, r), cwag_send.at[2, r], cwag_recv.at[2, r],
               right_p).wait_recv()
            mk(ccw_rows(3, r), ccwag_send.at[2, r], ccwag_recv.at[2, r],
               left_p).wait_recv()

        for rdma in started:
            rdma.wait_send()

    n_steps = PLANE - 1
    return pl.pallas_call(
        body,
        out_shape=jax.ShapeDtypeStruct((rows, cols), partial2d.dtype),
        in_specs=[
            pl.BlockSpec(memory_space=pltpu.VMEM),
            pl.BlockSpec(memory_space=pltpu.VMEM),
            pl.BlockSpec(memory_space=pltpu.VMEM),
        ],
        out_specs=pl.BlockSpec(memory_space=pltpu.VMEM),
        scratch_shapes=[
            pltpu.VMEM((rows, cols), partial2d.dtype),
            pltpu.VMEM((n_steps, HALF, cols), partial2d.dtype),
            pltpu.VMEM((n_steps, HALF, cols), partial2d.dtype),
            pltpu.VMEM((n_steps, 2, SUB, cols), partial2d.dtype),
            pltpu.SemaphoreType.DMA((n_steps, 2)),
            pltpu.SemaphoreType.DMA((n_steps, 2)),
            pltpu.SemaphoreType.DMA((n_steps, 2)),
            pltpu.SemaphoreType.DMA((n_steps, 2)),
            pltpu.SemaphoreType.DMA((n_steps, 2)),
            pltpu.SemaphoreType.DMA((n_steps, 2)),
            pltpu.SemaphoreType.DMA((n_steps, PLANE)),
            pltpu.SemaphoreType.DMA((n_steps, PLANE)),
            pltpu.SemaphoreType.DMA((n_steps, PLANE)),
            pltpu.SemaphoreType.DMA((n_steps, PLANE)),
        ],
        compiler_params=pltpu.CompilerParams(collective_id=collective_id),
    )(partial2d, base2d, gate)


def kernel(x, Wq, Wk, Wv, Wo, t_emb, W_mod, W_ff1, W_ff2):
    B, S, D = x.shape
    eps = 1e-5
    Dh = 96

    mod = t_emb @ W_mod
    sa, sha, ga, sm, shm, gm = jnp.split(mod, 6, axis=-1)

    def ln(h):
        m = h.mean(axis=-1, keepdims=True)
        v = h.var(axis=-1, keepdims=True)
        return (h - m) * lax.rsqrt(v + eps)

    x0 = x
    xa = ln(x0) * (1.0 + sa[:, None, :]) + sha[:, None, :]

    h_local = Wq.shape[1] // Dh
    Q = (xa @ Wq).reshape(B, S, h_local, Dh)
    K = (xa @ Wk).reshape(B, S, h_local, Dh)
    V = (xa @ Wv).reshape(B, S, h_local, Dh)
    scores = jnp.einsum("bihd,bjhd->bhij", Q, K) * (1.0 / (Dh ** 0.5))
    p = jax.nn.softmax(scores, axis=-1)
    o = jnp.einsum("bhij,bjhd->bihd", p, V).reshape(B, S, h_local * Dh)
    partial_attn = o @ Wo

    x1_2d = _ar_residual(
        partial_attn.reshape(B * S, D), x0.reshape(B * S, D), ga, 0
    )
    x1 = x1_2d.reshape(B, S, D)

    xm = ln(x1) * (1.0 + sm[:, None, :]) + shm[:, None, :]
    h = xm @ W_ff1
    h = h * jax.nn.sigmoid(h)
    partial_ff = h @ W_ff2

    out2d = _ar_residual(partial_ff.reshape(B * S, D), x1_2d, gm, 1)
    return out2d.reshape(B, S, D)


# device time: 135269 ns/iter; 1.9506x vs baseline; 1.0002x over previous
import jax
import jax.numpy as jnp
from jax import lax
from jax.experimental import pallas as pl
from jax.experimental.pallas import tpu as pltpu

N_DEV = 16
PLANE = 4
GROUP = 256
HALF = 128
SUB = 32


def _ar_residual(partial2d, base2d, gate, collective_id):
    rows, cols = partial2d.shape

    def body(p_ref, base_ref, gate_ref, out_ref,
             acc, pbufA, pbufB, zbuf,
             cw_send, ccw_send, z_send, dummy_sem, prs_send, zrs_send,
             cwrs_recv, ccwrs_recv, zrs_recv, zag_recv,
             cwag_recv, ccwag_recv):
        my = lax.axis_index("i")
        z = my // PLANE
        k = my % PLANE
        right_p = z * PLANE + (k + 1) % PLANE
        left_p = z * PLANE + (k + 3) % PLANE
        up = ((z + 1) % PLANE) * PLANE + k
        down = ((z + 3) % PLANE) * PLANE + k

        A_k = (k + 1) % PLANE
        B_k = (k + 3) % PLANE

        def arow(g, j):
            return g * GROUP + j * SUB

        def brow(g, j):
            return g * GROUP + HALF + j * SUB

        def j_of(c):
            return (z - c) % PLANE

        class Link:

            def __init__(self, sems, dev):
                self.sems = sems
                self.dev = dev
                self.i = 0
                self.q = []

            def send(self, src, dst, recv_sem):
                rdma = pltpu.make_async_remote_copy(
                    src_ref=src, dst_ref=dst,
                    send_sem=self.sems.at[self.i % 4],
                    recv_sem=recv_sem,
                    device_id=(self.dev,),
                    device_id_type=pl.DeviceIdType.MESH,
                )
                rdma.start()
                self.i += 1
                self.q.append(rdma)
                if len(self.q) > 2:
                    self.q.pop(0).wait_send()

            def drain(self):
                for r in self.q:
                    r.wait_send()
                self.q = []

        cw = Link(cw_send, right_p)
        ccw = Link(ccw_send, left_p)
        zl = Link(z_send, up)

        def wait_recv(dst, sem_recv, dev):
            pltpu.make_async_remote_copy(
                src_ref=dst, dst_ref=dst, send_sem=dummy_sem.at[0],
                recv_sem=sem_recv, device_id=(dev,),
                device_id_type=pl.DeviceIdType.MESH,
            ).wait_recv()

        barrier_sem = pltpu.get_barrier_semaphore()
        for nbr in (left_p, right_p, up, down):
            pl.semaphore_signal(
                barrier_sem, inc=1,
                device_id=(nbr,), device_id_type=pl.DeviceIdType.MESH,
            )
        pl.semaphore_wait(barrier_sem, 4)

        acc[...] = p_ref[...]

        for t in range(PLANE - 1):
            ga_s = (k - t) % PLANE
            gb_s = (k + t) % PLANE
            cw_rdma = pltpu.make_async_remote_copy(
                src_ref=acc.at[pl.ds(ga_s * GROUP, HALF)],
                dst_ref=pbufA.at[t],
                send_sem=prs_send.at[t, 0], recv_sem=cwrs_recv.at[t, 0],
                device_id=(right_p,), device_id_type=pl.DeviceIdType.MESH,
            )
            ccw_rdma = pltpu.make_async_remote_copy(
                src_ref=acc.at[pl.ds(gb_s * GROUP + HALF, HALF)],
                dst_ref=pbufB.at[t],
                send_sem=prs_send.at[t, 1], recv_sem=ccwrs_recv.at[t, 0],
                device_id=(left_p,), device_id_type=pl.DeviceIdType.MESH,
            )
            cw_rdma.start()
            ccw_rdma.start()
            cw_rdma.wait()
            ccw_rdma.wait()
            ga_r = (k - t - 1) % PLANE
            gb_r = (k + t + 1) % PLANE
            acc[pl.ds(ga_r * GROUP, HALF), :] += pbufA[t]
            acc[pl.ds(gb_r * GROUP + HALF, HALF), :] += pbufB[t]

        for s in range(PLANE - 1):
            j_s = (z - s) % PLANE
            za = pltpu.make_async_remote_copy(
                src_ref=acc.at[pl.ds(arow(A_k, j_s), SUB)],
                dst_ref=zbuf.at[s, 0],
                send_sem=zrs_send.at[s, 0], recv_sem=zrs_recv.at[s, 0],
                device_id=(up,), device_id_type=pl.DeviceIdType.MESH,
            )
            zb = pltpu.make_async_remote_copy(
                src_ref=acc.at[pl.ds(brow(B_k, j_s), SUB)],
                dst_ref=zbuf.at[s, 1],
                send_sem=zrs_send.at[s, 1], recv_sem=zrs_recv.at[s, 1],
                device_id=(up,), device_id_type=pl.DeviceIdType.MESH,
            )
            za.start()
            zb.start()
            za.wait()
            zb.wait()
            j_r = (z - s - 1) % PLANE
            acc[pl.ds(arow(A_k, j_r), SUB), :] += zbuf[s, 0]
            acc[pl.ds(brow(B_k, j_r), SUB), :] += zbuf[s, 1]
        j_own = (z + 1) % PLANE

        for off in (arow(A_k, j_own), brow(B_k, j_own)):
            b = off // 512
            g_row = gate_ref[pl.ds(b, 1), :]
            out_ref[pl.ds(off, SUB), :] = (
                base_ref[pl.ds(off, SUB), :] + g_row * acc[pl.ds(off, SUB), :]
            )

        def jag(r):
            return (z + 1 - r) % PLANE

        def out_at(rowoff):
            return out_ref.at[pl.ds(rowoff, SUB)]

        def cw_rows(t, r):
            return arow((k + 1 - t) % PLANE, jag(r))

        def ccw_rows(t, r):
            return brow((k + 3 + t) % PLANE, jag(r))

        zl.send(out_at(arow(A_k, jag(0))), out_at(arow(A_k, jag(0))),
                zag_recv.at[0, 0])
        zl.send(out_at(brow(B_k, jag(0))), out_at(brow(B_k, jag(0))),
                zag_recv.at[0, 1])
        cw.send(out_at(cw_rows(0, 0)), out_at(cw_rows(0, 0)),
                cwag_recv.at[0, 0])
        ccw.send(out_at(ccw_rows(0, 0)), out_at(ccw_rows(0, 0)),
                 ccwag_recv.at[0, 0])
        for s in range(1, PLANE):
            ja = jag(s)
            wait_recv(out_at(arow(A_k, ja)), zag_recv.at[s - 1, 0], up)
            wait_recv(out_at(brow(B_k, ja)), zag_recv.at[s - 1, 1], up)
            if s < PLANE - 1:
                zl.send(out_at(arow(A_k, ja)), out_at(arow(A_k, ja)),
                        zag_recv.at[s, 0])
                zl.send(out_at(brow(B_k, ja)), out_at(brow(B_k, ja)),
                        zag_recv.at[s, 1])
            cw.send(out_at(cw_rows(0, s)), out_at(cw_rows(0, s)),
                    cwag_recv.at[0, s])
            ccw.send(out_at(ccw_rows(0, s)), out_at(ccw_rows(0, s)),
                     ccwag_recv.at[0, s])
        for t in (1, 2):
            for r in range(PLANE):
                wait_recv(out_at(cw_rows(t, r)), cwag_recv.at[t - 1, r],
                          right_p)
                cw.send(out_at(cw_rows(t, r)), out_at(cw_rows(t, r)),
                        cwag_recv.at[t, r])
                wait_recv(out_at(ccw_rows(t, r)), ccwag_recv.at[t - 1, r],
                          left_p)
                ccw.send(out_at(ccw_rows(t, r)), out_at(ccw_rows(t, r)),
                         ccwag_recv.at[t, r])
        for r in range(PLANE):
            wait_recv(out_at(cw_rows(3, r)), cwag_recv.at[2, r], right_p)
            wait_recv(out_at(ccw_rows(3, r)), ccwag_recv.at[2, r], left_p)

        cw.drain()
        ccw.drain()
        zl.drain()

    n_steps = PLANE - 1
    sem4 = pltpu.SemaphoreType.DMA((n_steps, PLANE))
    sem2 = pltpu.SemaphoreType.DMA((n_steps, 2))
    return pl.pallas_call(
        body,
        out_shape=jax.ShapeDtypeStruct((rows, cols), partial2d.dtype),
        in_specs=[
            pl.BlockSpec(memory_space=pltpu.VMEM),
            pl.BlockSpec(memory_space=pltpu.VMEM),
            pl.BlockSpec(memory_space=pltpu.VMEM),
        ],
        out_specs=pl.BlockSpec(memory_space=pltpu.VMEM),
        scratch_shapes=[
            pltpu.VMEM((rows, cols), partial2d.dtype),
            pltpu.VMEM((n_steps, HALF, cols), partial2d.dtype),
            pltpu.VMEM((n_steps, HALF, cols), partial2d.dtype),
            pltpu.VMEM((n_steps, 2, SUB, cols), partial2d.dtype),
            pltpu.SemaphoreType.DMA((4,)),
            pltpu.SemaphoreType.DMA((4,)),
            pltpu.SemaphoreType.DMA((4,)),
            pltpu.SemaphoreType.DMA((1,)),
            sem2,
            sem2,
            sem4,
            sem4,
            sem2,
            sem2,
            sem4,
            sem4,
        ],
        compiler_params=pltpu.CompilerParams(collective_id=collective_id),
    )(partial2d, base2d, gate)


def kernel(x, Wq, Wk, Wv, Wo, t_emb, W_mod, W_ff1, W_ff2):
    B, S, D = x.shape
    eps = 1e-5
    Dh = 96

    mod = t_emb @ W_mod
    sa, sha, ga, sm, shm, gm = jnp.split(mod, 6, axis=-1)

    def ln(h):
        m = h.mean(axis=-1, keepdims=True)
        v = h.var(axis=-1, keepdims=True)
        return (h - m) * lax.rsqrt(v + eps)

    x0 = x
    xa = ln(x0) * (1.0 + sa[:, None, :]) + sha[:, None, :]

    h_local = Wq.shape[1] // Dh
    Q = (xa @ Wq).reshape(B, S, h_local, Dh)
    K = (xa @ Wk).reshape(B, S, h_local, Dh)
    V = (xa @ Wv).reshape(B, S, h_local, Dh)
    scores = jnp.einsum("bihd,bjhd->bhij", Q, K) * (1.0 / (Dh ** 0.5))
    p = jax.nn.softmax(scores, axis=-1)
    o = jnp.einsum("bhij,bjhd->bihd", p, V).reshape(B, S, h_local * Dh)
    partial_attn = o @ Wo

    x1_2d = _ar_residual(
        partial_attn.reshape(B * S, D), x0.reshape(B * S, D), ga, 0
    )
    x1 = x1_2d.reshape(B, S, D)

    xm = ln(x1) * (1.0 + sm[:, None, :]) + shm[:, None, :]
    h = xm @ W_ff1
    h = h * jax.nn.sigmoid(h)
    partial_ff = h @ W_ff2

    out2d = _ar_residual(partial_ff.reshape(B * S, D), x1_2d, gm, 1)
    return out2d.reshape(B, S, D)


# device time: 120591 ns/iter; 2.1881x vs baseline; 1.1217x over previous
import jax
import jax.numpy as jnp
from jax import lax
from jax.experimental import pallas as pl
from jax.experimental.pallas import tpu as pltpu

N_DEV = 16
PLANE = 4
GROUP = 256
HALF = 128
SUB = 32


def _ar_residual(partial2d, base2d, gate, collective_id):
    rows, cols = partial2d.shape

    def body(p_ref, base_ref, gate_ref, out_ref,
             pbufA, pbufB, zbuf, sbufA, sbufB, zsbuf,
             cw_send, ccw_send, z_send, dummy_sem,
             cwrs_recv, ccwrs_recv, zrs_recv, zag_recv,
             cwag_recv, ccwag_recv):
        my = lax.axis_index("i")
        z = my // PLANE
        k = my % PLANE
        right_p = z * PLANE + (k + 1) % PLANE
        left_p = z * PLANE + (k + 3) % PLANE
        up = ((z + 1) % PLANE) * PLANE + k
        down = ((z + 3) % PLANE) * PLANE + k

        A_k = (k + 1) % PLANE
        B_k = (k + 3) % PLANE

        def arow(g, j):
            return g * GROUP + j * SUB

        def brow(g, j):
            return g * GROUP + HALF + j * SUB

        def j_of(c):
            return (z - c) % PLANE

        class Link:

            def __init__(self, sems, dev):
                self.sems = sems
                self.dev = dev
                self.i = 0
                self.q = []

            def send(self, src, dst, recv_sem):
                rdma = pltpu.make_async_remote_copy(
                    src_ref=src, dst_ref=dst,
                    send_sem=self.sems.at[self.i % 4],
                    recv_sem=recv_sem,
                    device_id=(self.dev,),
                    device_id_type=pl.DeviceIdType.MESH,
                )
                rdma.start()
                self.i += 1
                self.q.append(rdma)
                if len(self.q) > 2:
                    self.q.pop(0).wait_send()

            def drain(self):
                for r in self.q:
                    r.wait_send()
                self.q = []

        cw = Link(cw_send, right_p)
        ccw = Link(ccw_send, left_p)
        zl = Link(z_send, up)

        def wait_recv(dst, sem_recv, dev):
            pltpu.make_async_remote_copy(
                src_ref=dst, dst_ref=dst, send_sem=dummy_sem.at[0],
                recv_sem=sem_recv, device_id=(dev,),
                device_id_type=pl.DeviceIdType.MESH,
            ).wait_recv()

        barrier_sem = pltpu.get_barrier_semaphore()
        for nbr in (left_p, right_p, up, down):
            pl.semaphore_signal(
                barrier_sem, inc=1,
                device_id=(nbr,), device_id_type=pl.DeviceIdType.MESH,
            )
        pl.semaphore_wait(barrier_sem, 4)

        for c in range(PLANE):
            cw.send(p_ref.at[pl.ds(arow(k, j_of(c)), SUB)], pbufA.at[c],
                    cwrs_recv.at[0, c])
            ccw.send(p_ref.at[pl.ds(brow(k, j_of(c)), SUB)], pbufB.at[c],
                     ccwrs_recv.at[0, c])
        for t in (1, 2):
            for c in range(PLANE):
                j = j_of(c)
                idx = (t - 1) * PLANE + c
                ga = (k - t) % PLANE
                wait_recv(pbufA.at[idx], cwrs_recv.at[t - 1, c], right_p)
                sbufA[idx] = p_ref[pl.ds(arow(ga, j), SUB), :] + pbufA[idx]
                cw.send(sbufA.at[idx], pbufA.at[t * PLANE + c],
                        cwrs_recv.at[t, c])
                gb = (k + t) % PLANE
                wait_recv(pbufB.at[idx], ccwrs_recv.at[t - 1, c], left_p)
                sbufB[idx] = p_ref[pl.ds(brow(gb, j), SUB), :] + pbufB[idx]
                ccw.send(sbufB.at[idx], pbufB.at[t * PLANE + c],
                         ccwrs_recv.at[t, c])

        for c in range(PLANE):
            j = j_of(c)
            i2 = 2 * PLANE + c
            wait_recv(pbufA.at[i2], cwrs_recv.at[2, c], right_p)
            wait_recv(pbufB.at[i2], ccwrs_recv.at[2, c], left_p)
            if c >= 1:
                wait_recv(zbuf.at[c - 1, 0], zrs_recv.at[c - 1, 0], up)
                wait_recv(zbuf.at[c - 1, 1], zrs_recv.at[c - 1, 1], up)
            pa = p_ref[pl.ds(arow(A_k, j), SUB), :] + pbufA[i2]
            pb = p_ref[pl.ds(brow(B_k, j), SUB), :] + pbufB[i2]
            if c >= 1:
                pa = pa + zbuf[c - 1, 0]
                pb = pb + zbuf[c - 1, 1]
            if c <= 2:
                zsbuf[c, 0] = pa
                zsbuf[c, 1] = pb
                zl.send(zsbuf.at[c, 0], zbuf.at[c, 0], zrs_recv.at[c, 0])
                zl.send(zsbuf.at[c, 1], zbuf.at[c, 1], zrs_recv.at[c, 1])
            else:
                for off, val in ((arow(A_k, j), pa), (brow(B_k, j), pb)):
                    g_row = gate_ref[pl.ds(off // 512, 1), :]
                    out_ref[pl.ds(off, SUB), :] = (
                        base_ref[pl.ds(off, SUB), :] + g_row * val
                    )
        j_own = (z + 1) % PLANE

        def jag(r):
            return (z + 1 - r) % PLANE

        def out_at(rowoff):
            return out_ref.at[pl.ds(rowoff, SUB)]

        def cw_rows(t, r):
            return arow((k + 1 - t) % PLANE, jag(r))

        def ccw_rows(t, r):
            return brow((k + 3 + t) % PLANE, jag(r))

        zl.send(out_at(arow(A_k, jag(0))), out_at(arow(A_k, jag(0))),
                zag_recv.at[0, 0])
        zl.send(out_at(brow(B_k, jag(0))), out_at(brow(B_k, jag(0))),
                zag_recv.at[0, 1])
        cw.send(out_at(cw_rows(0, 0)), out_at(cw_rows(0, 0)),
                cwag_recv.at[0, 0])
        ccw.send(out_at(ccw_rows(0, 0)), out_at(ccw_rows(0, 0)),
                 ccwag_recv.at[0, 0])
        for s in range(1, PLANE):
            ja = jag(s)
            wait_recv(out_at(arow(A_k, ja)), zag_recv.at[s - 1, 0], up)
            wait_recv(out_at(brow(B_k, ja)), zag_recv.at[s - 1, 1], up)
            if s < PLANE - 1:
                zl.send(out_at(arow(A_k, ja)), out_at(arow(A_k, ja)),
                        zag_recv.at[s, 0])
                zl.send(out_at(brow(B_k, ja)), out_at(brow(B_k, ja)),
                        zag_recv.at[s, 1])
            cw.send(out_at(cw_rows(0, s)), out_at(cw_rows(0, s)),
                    cwag_recv.at[0, s])
            ccw.send(out_at(ccw_rows(0, s)), out_at(ccw_rows(0, s)),
                     ccwag_recv.at[0, s])
        for t in (1, 2):
            for r in range(PLANE):
                wait_recv(out_at(cw_rows(t, r)), cwag_recv.at[t - 1, r],
                          right_p)
                cw.send(out_at(cw_rows(t, r)), out_at(cw_rows(t, r)),
                        cwag_recv.at[t, r])
                wait_recv(out_at(ccw_rows(t, r)), ccwag_recv.at[t - 1, r],
                          left_p)
                ccw.send(out_at(ccw_rows(t, r)), out_at(ccw_rows(t, r)),
                         ccwag_recv.at[t, r])
        for r in range(PLANE):
            wait_recv(out_at(cw_rows(3, r)), cwag_recv.at[2, r], right_p)
            wait_recv(out_at(ccw_rows(3, r)), ccwag_recv.at[2, r], left_p)

        cw.drain()
        ccw.drain()
        zl.drain()

    n_steps = PLANE - 1
    sem4 = pltpu.SemaphoreType.DMA((n_steps, PLANE))
    sem2 = pltpu.SemaphoreType.DMA((n_steps, 2))
    return pl.pallas_call(
        body,
        out_shape=jax.ShapeDtypeStruct((rows, cols), partial2d.dtype),
        in_specs=[
            pl.BlockSpec(memory_space=pltpu.VMEM),
            pl.BlockSpec(memory_space=pltpu.VMEM),
            pl.BlockSpec(memory_space=pltpu.VMEM),
        ],
        out_specs=pl.BlockSpec(memory_space=pltpu.VMEM),
        scratch_shapes=[
            pltpu.VMEM((n_steps * PLANE, SUB, cols), partial2d.dtype),
            pltpu.VMEM((n_steps * PLANE, SUB, cols), partial2d.dtype),
            pltpu.VMEM((n_steps, 2, SUB, cols), partial2d.dtype),
            pltpu.VMEM((2 * PLANE, SUB, cols), partial2d.dtype),
            pltpu.VMEM((2 * PLANE, SUB, cols), partial2d.dtype),
            pltpu.VMEM((n_steps, 2, SUB, cols), partial2d.dtype),
            pltpu.SemaphoreType.DMA((4,)),
            pltpu.SemaphoreType.DMA((4,)),
            pltpu.SemaphoreType.DMA((4,)),
            pltpu.SemaphoreType.DMA((1,)),
            sem4,
            sem4,
            sem2,
            sem2,
            sem4,
            sem4,
        ],
        compiler_params=pltpu.CompilerParams(collective_id=collective_id),
    )(partial2d, base2d, gate)


def kernel(x, Wq, Wk, Wv, Wo, t_emb, W_mod, W_ff1, W_ff2):
    B, S, D = x.shape
    eps = 1e-5
    Dh = 96

    mod = t_emb @ W_mod
    sa, sha, ga, sm, shm, gm = jnp.split(mod, 6, axis=-1)

    def ln(h):
        m = h.mean(axis=-1, keepdims=True)
        v = h.var(axis=-1, keepdims=True)
        return (h - m) * lax.rsqrt(v + eps)

    x0 = x
    xa = ln(x0) * (1.0 + sa[:, None, :]) + sha[:, None, :]

    h_local = Wq.shape[1] // Dh
    Q = (xa @ Wq).reshape(B, S, h_local, Dh)
    K = (xa @ Wk).reshape(B, S, h_local, Dh)
    V = (xa @ Wv).reshape(B, S, h_local, Dh)
    scores = jnp.einsum("bihd,bjhd->bhij", Q, K) * (1.0 / (Dh ** 0.5))
    p = jax.nn.softmax(scores, axis=-1)
    o = jnp.einsum("bhij,bjhd->bihd", p, V).reshape(B, S, h_local * Dh)
    partial_attn = o @ Wo

    x1_2d = _ar_residual(
        partial_attn.reshape(B * S, D), x0.reshape(B * S, D), ga, 0
    )
    x1 = x1_2d.reshape(B, S, D)

    xm = ln(x1) * (1.0 + sm[:, None, :]) + shm[:, None, :]
    h = xm @ W_ff1
    h = h * jax.nn.sigmoid(h)
    partial_ff = h @ W_ff2

    out2d = _ar_residual(partial_ff.reshape(B * S, D), x1_2d, gm, 1)
    return out2d.reshape(B, S, D)


# device time: 109748 ns/iter; 2.4042x vs baseline; 1.0988x over previous
import jax
import jax.numpy as jnp
from jax import lax
from jax.experimental import pallas as pl
from jax.experimental.pallas import tpu as pltpu

N_DEV = 16
PLANE = 4
GROUP = 256
HALF = 128
SUB = 32


def _ar_residual(partial2d, base2d, gate, collective_id):
    rows, cols = partial2d.shape

    def body(p_ref, base_ref, gate_ref, out_ref,
             pbufA, pbufB, zbuf, sbufA, sbufB, zsbuf,
             cw_send, ccw_send, z_send, z2_send, dummy_sem,
             cwrs_recv, ccwrs_recv, zrs_recv, zag_recv,
             cwag_recv, ccwag_recv):
        my = lax.axis_index("i")
        z = my // PLANE
        k = my % PLANE
        right_p = z * PLANE + (k + 1) % PLANE
        left_p = z * PLANE + (k + 3) % PLANE
        up = ((z + 1) % PLANE) * PLANE + k
        down = ((z + 3) % PLANE) * PLANE + k

        A_k = (k + 1) % PLANE
        B_k = (k + 3) % PLANE

        def arow(g, j):
            return g * GROUP + j * SUB

        def brow(g, j):
            return g * GROUP + HALF + j * SUB

        def j_of(c):
            return (z - c) % PLANE

        class Link:

            def __init__(self, sems, dev):
                self.sems = sems
                self.dev = dev
                self.i = 0
                self.q = []

            def send(self, src, dst, recv_sem):
                rdma = pltpu.make_async_remote_copy(
                    src_ref=src, dst_ref=dst,
                    send_sem=self.sems.at[self.i % 4],
                    recv_sem=recv_sem,
                    device_id=(self.dev,),
                    device_id_type=pl.DeviceIdType.MESH,
                )
                rdma.start()
                self.i += 1
                self.q.append(rdma)
                if len(self.q) > 2:
                    self.q.pop(0).wait_send()

            def drain(self):
                for r in self.q:
                    r.wait_send()
                self.q = []

        cw = Link(cw_send, right_p)
        ccw = Link(ccw_send, left_p)
        zl = Link(z_send, up)
        zdl = Link(z2_send, down)

        def wait_recv(dst, sem_recv, dev):
            pltpu.make_async_remote_copy(
                src_ref=dst, dst_ref=dst, send_sem=dummy_sem.at[0],
                recv_sem=sem_recv, device_id=(dev,),
                device_id_type=pl.DeviceIdType.MESH,
            ).wait_recv()

        barrier_sem = pltpu.get_barrier_semaphore()
        for nbr in (left_p, right_p, up, down):
            pl.semaphore_signal(
                barrier_sem, inc=1,
                device_id=(nbr,), device_id_type=pl.DeviceIdType.MESH,
            )
        pl.semaphore_wait(barrier_sem, 4)

        for c in range(PLANE):
            cw.send(p_ref.at[pl.ds(arow(k, j_of(c)), SUB)], pbufA.at[c],
                    cwrs_recv.at[0, c])
            ccw.send(p_ref.at[pl.ds(brow(k, j_of(c)), SUB)], pbufB.at[c],
                     ccwrs_recv.at[0, c])
        for t in (1, 2):
            for c in range(PLANE):
                j = j_of(c)
                idx = (t - 1) * PLANE + c
                ga = (k - t) % PLANE
                wait_recv(pbufA.at[idx], cwrs_recv.at[t - 1, c], right_p)
                sbufA[idx] = p_ref[pl.ds(arow(ga, j), SUB), :] + pbufA[idx]
                cw.send(sbufA.at[idx], pbufA.at[t * PLANE + c],
                        cwrs_recv.at[t, c])
                gb = (k + t) % PLANE
                wait_recv(pbufB.at[idx], ccwrs_recv.at[t - 1, c], left_p)
                sbufB[idx] = p_ref[pl.ds(brow(gb, j), SUB), :] + pbufB[idx]
                ccw.send(sbufB.at[idx], pbufB.at[t * PLANE + c],
                         ccwrs_recv.at[t, c])

        for c in range(PLANE):
            j = j_of(c)
            i2 = 2 * PLANE + c
            wait_recv(pbufA.at[i2], cwrs_recv.at[2, c], right_p)
            wait_recv(pbufB.at[i2], ccwrs_recv.at[2, c], left_p)
            if c >= 1:
                wait_recv(zbuf.at[c - 1, 0], zrs_recv.at[c - 1, 0], up)
                wait_recv(zbuf.at[c - 1, 1], zrs_recv.at[c - 1, 1], up)
            pa = p_ref[pl.ds(arow(A_k, j), SUB), :] + pbufA[i2]
            pb = p_ref[pl.ds(brow(B_k, j), SUB), :] + pbufB[i2]
            if c >= 1:
                pa = pa + zbuf[c - 1, 0]
                pb = pb + zbuf[c - 1, 1]
            if c <= 2:
                zsbuf[c, 0] = pa
                zsbuf[c, 1] = pb
                zl.send(zsbuf.at[c, 0], zbuf.at[c, 0], zrs_recv.at[c, 0])
                zl.send(zsbuf.at[c, 1], zbuf.at[c, 1], zrs_recv.at[c, 1])
            else:
                for off, val in ((arow(A_k, j), pa), (brow(B_k, j), pb)):
                    g_row = gate_ref[pl.ds(off // 512, 1), :]
                    out_ref[pl.ds(off, SUB), :] = (
                        base_ref[pl.ds(off, SUB), :] + g_row * val
                    )
        j_own = (z + 1) % PLANE

        def jag(r):
            return (z + 1 + (0, 3, 1, 2)[r]) % PLANE

        def out_at(rowoff):
            return out_ref.at[pl.ds(rowoff, SUB)]

        def plane_own_sends(r):
            a = out_at(arow(A_k, jag(r)))
            b = out_at(brow(B_k, jag(r)))
            cw.send(a, a, cwag_recv.at[0, r])
            ccw.send(a, a, cwag_recv.at[2, r])
            ccw.send(b, b, ccwag_recv.at[0, r])
            cw.send(b, b, ccwag_recv.at[2, r])

        own_a = out_at(arow(A_k, jag(0)))
        own_b = out_at(brow(B_k, jag(0)))
        zl.send(own_a, own_a, zag_recv.at[0, 0])
        zdl.send(own_a, own_a, zag_recv.at[1, 0])
        zl.send(own_b, own_b, zag_recv.at[0, 1])
        zdl.send(own_b, own_b, zag_recv.at[1, 1])
        plane_own_sends(0)
        wait_recv(out_at(arow(A_k, jag(1))), zag_recv.at[0, 0], down)
        wait_recv(out_at(brow(B_k, jag(1))), zag_recv.at[0, 1], down)
        zl.send(out_at(arow(A_k, jag(1))), out_at(arow(A_k, jag(1))),
                zag_recv.at[2, 0])
        plane_own_sends(1)
        wait_recv(out_at(arow(A_k, jag(2))), zag_recv.at[1, 0], up)
        wait_recv(out_at(brow(B_k, jag(2))), zag_recv.at[1, 1], up)
        zdl.send(out_at(brow(B_k, jag(2))), out_at(brow(B_k, jag(2))),
                 zag_recv.at[2, 1])
        plane_own_sends(2)
        wait_recv(out_at(arow(A_k, jag(3))), zag_recv.at[2, 0], down)
        wait_recv(out_at(brow(B_k, jag(3))), zag_recv.at[2, 1], up)
        plane_own_sends(3)
        for r in range(PLANE):
            wait_recv(out_at(arow(k, jag(r))), cwag_recv.at[0, r], left_p)
            cw.send(out_at(arow(k, jag(r))), out_at(arow(k, jag(r))),
                    cwag_recv.at[1, r])
            wait_recv(out_at(brow(k, jag(r))), ccwag_recv.at[0, r], right_p)
            ccw.send(out_at(brow(k, jag(r))), out_at(brow(k, jag(r))),
                     ccwag_recv.at[1, r])
        for r in range(PLANE):
            wait_recv(out_at(arow((k + 3) % PLANE, jag(r))),
                      cwag_recv.at[1, r], left_p)
            wait_recv(out_at(arow((k + 2) % PLANE, jag(r))),
                      cwag_recv.at[2, r], right_p)
            wait_recv(out_at(brow((k + 1) % PLANE, jag(r))),
                      ccwag_recv.at[1, r], right_p)
            wait_recv(out_at(brow((k + 2) % PLANE, jag(r))),
                      ccwag_recv.at[2, r], left_p)

        cw.drain()
        ccw.drain()
        zl.drain()
        zdl.drain()

    n_steps = PLANE - 1
    sem4 = pltpu.SemaphoreType.DMA((n_steps, PLANE))
    sem2 = pltpu.SemaphoreType.DMA((n_steps, 2))
    return pl.pallas_call(
        body,
        out_shape=jax.ShapeDtypeStruct((rows, cols), partial2d.dtype),
        in_specs=[
            pl.BlockSpec(memory_space=pltpu.VMEM),
            pl.BlockSpec(memory_space=pltpu.VMEM),
            pl.BlockSpec(memory_space=pltpu.VMEM),
        ],
        out_specs=pl.BlockSpec(memory_space=pltpu.VMEM),
        scratch_shapes=[
            pltpu.VMEM((n_steps * PLANE, SUB, cols), partial2d.dtype),
            pltpu.VMEM((n_steps * PLANE, SUB, cols), partial2d.dtype),
            pltpu.VMEM((n_steps, 2, SUB, cols), partial2d.dtype),
            pltpu.VMEM((2 * PLANE, SUB, cols), partial2d.dtype),
            pltpu.VMEM((2 * PLANE, SUB, cols), partial2d.dtype),
            pltpu.VMEM((n_steps, 2, SUB, cols), partial2d.dtype),
            pltpu.SemaphoreType.DMA((4,)),
            pltpu.SemaphoreType.DMA((4,)),
            pltpu.SemaphoreType.DMA((4,)),
            pltpu.SemaphoreType.DMA((4,)),
            pltpu.SemaphoreType.DMA((1,)),
            sem4,
            sem4,
            sem2,
            sem2,
            sem4,
            sem4,
        ],
        compiler_params=pltpu.CompilerParams(collective_id=collective_id),
    )(partial2d, base2d, gate)


def kernel(x, Wq, Wk, Wv, Wo, t_emb, W_mod, W_ff1, W_ff2):
    B, S, D = x.shape
    eps = 1e-5
    Dh = 96

    mod = t_emb @ W_mod
    sa, sha, ga, sm, shm, gm = jnp.split(mod, 6, axis=-1)

    def ln(h):
        m = h.mean(axis=-1, keepdims=True)
        v = h.var(axis=-1, keepdims=True)
        return (h - m) * lax.rsqrt(v + eps)

    x0 = x
    xa = ln(x0) * (1.0 + sa[:, None, :]) + sha[:, None, :]

    h_local = Wq.shape[1] // Dh
    Q = (xa @ Wq).reshape(B, S, h_local, Dh)
    K = (xa @ Wk).reshape(B, S, h_local, Dh)
    V = (xa @ Wv).reshape(B, S, h_local, Dh)
    scores = jnp.einsum("bihd,bjhd->bhij", Q, K) * (1.0 / (Dh ** 0.5))
    p = jax.nn.softmax(scores, axis=-1)
    o = jnp.einsum("bhij,bjhd->bihd", p, V).reshape(B, S, h_local * Dh)
    partial_attn = o @ Wo

    x1_2d = _ar_residual(
        partial_attn.reshape(B * S, D), x0.reshape(B * S, D), ga, 0
    )
    x1 = x1_2d.reshape(B, S, D)

    xm = ln(x1) * (1.0 + sm[:, None, :]) + shm[:, None, :]
    h = xm @ W_ff1
    h = h * jax.nn.sigmoid(h)
    partial_ff = h @ W_ff2

    out2d = _ar_residual(partial_ff.reshape(B * S, D), x1_2d, gm, 1)
    return out2d.reshape(B, S, D)


# device time: 107922 ns/iter; 2.4449x vs baseline; 1.0169x over previous
import jax
import jax.numpy as jnp
from jax import lax
from jax.experimental import pallas as pl
from jax.experimental.pallas import tpu as pltpu

N_DEV = 16
PLANE = 4
GROUP = 256
HALF = 128
SUB = 32


def _ar_residual(partial2d, base2d, gate, collective_id):
    rows, cols = partial2d.shape

    def body(p_ref, base_ref, gate_ref, out_ref,
             pbufA, pbufB, zbuf, sbufA, sbufB, zsbuf,
             cw_send, ccw_send, z_send, z2_send, dummy_sem,
             cwrs_recv, ccwrs_recv, zrs_recv, zag_recv,
             cwag_recv, ccwag_recv):
        my = lax.axis_index("i")
        z = my // PLANE
        k = my % PLANE
        right_p = z * PLANE + (k + 1) % PLANE
        left_p = z * PLANE + (k + 3) % PLANE
        up = ((z + 1) % PLANE) * PLANE + k
        down = ((z + 3) % PLANE) * PLANE + k

        A_k = (k + 1) % PLANE
        B_k = (k + 3) % PLANE

        def arow(g, j):
            return g * GROUP + j * SUB

        def brow(g, j):
            return g * GROUP + HALF + j * SUB

        def j_of(c):
            return (z - c) % PLANE

        class Link:

            def __init__(self, sems, dev):
                self.sems = sems
                self.dev = dev
                self.i = 0
                self.q = []

            def send(self, src, dst, recv_sem):
                rdma = pltpu.make_async_remote_copy(
                    src_ref=src, dst_ref=dst,
                    send_sem=self.sems.at[self.i % 4],
                    recv_sem=recv_sem,
                    device_id=(self.dev,),
                    device_id_type=pl.DeviceIdType.MESH,
                )
                rdma.start()
                self.i += 1
                self.q.append(rdma)
                if len(self.q) > 2:
                    self.q.pop(0).wait_send()

            def drain(self):
                for r in self.q:
                    r.wait_send()
                self.q = []

        cw = Link(cw_send, right_p)
        ccw = Link(ccw_send, left_p)
        zl = Link(z_send, up)
        zdl = Link(z2_send, down)

        def wait_recv(dst, sem_recv, dev):
            pltpu.make_async_remote_copy(
                src_ref=dst, dst_ref=dst, send_sem=dummy_sem.at[0],
                recv_sem=sem_recv, device_id=(dev,),
                device_id_type=pl.DeviceIdType.MESH,
            ).wait_recv()

        barrier_sem = pltpu.get_barrier_semaphore()
        for nbr in (left_p, right_p, up, down):
            pl.semaphore_signal(
                barrier_sem, inc=1,
                device_id=(nbr,), device_id_type=pl.DeviceIdType.MESH,
            )
        pl.semaphore_wait(barrier_sem, 4)

        for c in range(PLANE):
            cw.send(p_ref.at[pl.ds(arow(k, j_of(c)), SUB)], pbufA.at[c],
                    cwrs_recv.at[0, c])
            ccw.send(p_ref.at[pl.ds(brow(k, j_of(c)), SUB)], pbufB.at[c],
                     ccwrs_recv.at[0, c])
        for t in (1, 2):
            for c in range(PLANE):
                j = j_of(c)
                idx = (t - 1) * PLANE + c
                ga = (k - t) % PLANE
                wait_recv(pbufA.at[idx], cwrs_recv.at[t - 1, c], right_p)
                sbufA[idx] = p_ref[pl.ds(arow(ga, j), SUB), :] + pbufA[idx]
                cw.send(sbufA.at[idx], pbufA.at[t * PLANE + c],
                        cwrs_recv.at[t, c])
                gb = (k + t) % PLANE
                wait_recv(pbufB.at[idx], ccwrs_recv.at[t - 1, c], left_p)
                sbufB[idx] = p_ref[pl.ds(brow(gb, j), SUB), :] + pbufB[idx]
                ccw.send(sbufB.at[idx], pbufB.at[t * PLANE + c],
                         ccwrs_recv.at[t, c])

        for c in range(PLANE):
            j = j_of(c)
            i2 = 2 * PLANE + c
            wait_recv(pbufA.at[i2], cwrs_recv.at[2, c], right_p)
            wait_recv(pbufB.at[i2], ccwrs_recv.at[2, c], left_p)
            pa = p_ref[pl.ds(arow(A_k, j), SUB), :] + pbufA[i2]
            pb = p_ref[pl.ds(brow(B_k, j), SUB), :] + pbufB[i2]
            if c == 0:
                zsbuf[0, 0] = pa
                zdl.send(zsbuf.at[0, 0], zbuf.at[1, 0], zrs_recv.at[1, 0])
            elif c == 1:
                zsbuf[1, 0] = pa
                zl.send(zsbuf.at[1, 0], zbuf.at[0, 0], zrs_recv.at[0, 0])
                zsbuf[1, 1] = pb
                zdl.send(zsbuf.at[1, 1], zbuf.at[0, 1], zrs_recv.at[0, 1])
            elif c == 2:
                wait_recv(zbuf.at[0, 0], zrs_recv.at[0, 0], down)
                zsbuf[2, 0] = pa + zbuf[0, 0]
                zl.send(zsbuf.at[2, 0], zbuf.at[2, 0], zrs_recv.at[2, 0])
                zsbuf[2, 1] = pb
                zl.send(zsbuf.at[2, 1], zbuf.at[1, 1], zrs_recv.at[1, 1])
                wait_recv(zbuf.at[0, 1], zrs_recv.at[0, 1], up)
                zsbuf[0, 1] = (
                    p_ref[pl.ds(brow(B_k, z), SUB), :]
                    + pbufB[2 * PLANE + 0] + zbuf[0, 1]
                )
                zdl.send(zsbuf.at[0, 1], zbuf.at[2, 1], zrs_recv.at[2, 1])
            else:
                wait_recv(zbuf.at[1, 0], zrs_recv.at[1, 0], up)
                wait_recv(zbuf.at[2, 0], zrs_recv.at[2, 0], down)
                wait_recv(zbuf.at[1, 1], zrs_recv.at[1, 1], down)
                wait_recv(zbuf.at[2, 1], zrs_recv.at[2, 1], up)
                va = pa + zbuf[1, 0] + zbuf[2, 0]
                vb = pb + zbuf[1, 1] + zbuf[2, 1]
                for off, val in ((arow(A_k, j), va), (brow(B_k, j), vb)):
                    g_row = gate_ref[pl.ds(off // 512, 1), :]
                    out_ref[pl.ds(off, SUB), :] = (
                        base_ref[pl.ds(off, SUB), :] + g_row * val
                    )
        j_own = (z + 1) % PLANE

        def jag(r):
            return (z + 1 + (0, 3, 1, 2)[r]) % PLANE

        def out_at(rowoff):
            return out_ref.at[pl.ds(rowoff, SUB)]

        def plane_own_sends(r):
            a = out_at(arow(A_k, jag(r)))
            b = out_at(brow(B_k, jag(r)))
            cw.send(a, a, cwag_recv.at[0, r])
            ccw.send(a, a, cwag_recv.at[2, r])
            ccw.send(b, b, ccwag_recv.at[0, r])
            cw.send(b, b, ccwag_recv.at[2, r])

        own_a = out_at(arow(A_k, jag(0)))
        own_b = out_at(brow(B_k, jag(0)))
        zl.send(own_a, own_a, zag_recv.at[0, 0])
        zdl.send(own_a, own_a, zag_recv.at[1, 0])
        zl.send(own_b, own_b, zag_recv.at[0, 1])
        zdl.send(own_b, own_b, zag_recv.at[1, 1])
        plane_own_sends(0)
        wait_recv(out_at(arow(A_k, jag(1))), zag_recv.at[0, 0], down)
        wait_recv(out_at(brow(B_k, jag(1))), zag_recv.at[0, 1], down)
        zl.send(out_at(arow(A_k, jag(1))), out_at(arow(A_k, jag(1))),
                zag_recv.at[2, 0])
        plane_own_sends(1)
        wait_recv(out_at(arow(A_k, jag(2))), zag_recv.at[1, 0], up)
        wait_recv(out_at(brow(B_k, jag(2))), zag_recv.at[1, 1], up)
        zdl.send(out_at(brow(B_k, jag(2))), out_at(brow(B_k, jag(2))),
                 zag_recv.at[2, 1])
        plane_own_sends(2)
        wait_recv(out_at(arow(A_k, jag(3))), zag_recv.at[2, 0], down)
        wait_recv(out_at(brow(B_k, jag(3))), zag_recv.at[2, 1], up)
        plane_own_sends(3)
        for r in range(PLANE):
            wait_recv(out_at(arow(k, jag(r))), cwag_recv.at[0, r], left_p)
            cw.send(out_at(arow(k, jag(r))), out_at(arow(k, jag(r))),
                    cwag_recv.at[1, r])
            wait_recv(out_at(brow(k, jag(r))), ccwag_recv.at[0, r], right_p)
            ccw.send(out_at(brow(k, jag(r))), out_at(brow(k, jag(r))),
                     ccwag_recv.at[1, r])
        for r in range(PLANE):
            wait_recv(out_at(arow((k + 3) % PLANE, jag(r))),
                      cwag_recv.at[1, r], left_p)
            wait_recv(out_at(arow((k + 2) % PLANE, jag(r))),
                      cwag_recv.at[2, r], right_p)
            wait_recv(out_at(brow((k + 1) % PLANE, jag(r))),
                      ccwag_recv.at[1, r], right_p)
            wait_recv(out_at(brow((k + 2) % PLANE, jag(r))),
                      ccwag_recv.at[2, r], left_p)

        cw.drain()
        ccw.drain()
        zl.drain()
        zdl.drain()

    n_steps = PLANE - 1
    sem4 = pltpu.SemaphoreType.DMA((n_steps, PLANE))
    sem2 = pltpu.SemaphoreType.DMA((n_steps, 2))
    return pl.pallas_call(
        body,
        out_shape=jax.ShapeDtypeStruct((rows, cols), partial2d.dtype),
        in_specs=[
            pl.BlockSpec(memory_space=pltpu.VMEM),
            pl.BlockSpec(memory_space=pltpu.VMEM),
            pl.BlockSpec(memory_space=pltpu.VMEM),
        ],
        out_specs=pl.BlockSpec(memory_space=pltpu.VMEM),
        scratch_shapes=[
            pltpu.VMEM((n_steps * PLANE, SUB, cols), partial2d.dtype),
            pltpu.VMEM((n_steps * PLANE, SUB, cols), partial2d.dtype),
            pltpu.VMEM((n_steps, 2, SUB, cols), partial2d.dtype),
            pltpu.VMEM((2 * PLANE, SUB, cols), partial2d.dtype),
            pltpu.VMEM((2 * PLANE, SUB, cols), partial2d.dtype),
            pltpu.VMEM((n_steps, 2, SUB, cols), partial2d.dtype),
            pltpu.SemaphoreType.DMA((4,)),
            pltpu.SemaphoreType.DMA((4,)),
            pltpu.SemaphoreType.DMA((4,)),
            pltpu.SemaphoreType.DMA((4,)),
            pltpu.SemaphoreType.DMA((1,)),
            sem4,
            sem4,
            sem2,
            sem2,
            sem4,
            sem4,
        ],
        compiler_params=pltpu.CompilerParams(collective_id=collective_id),
    )(partial2d, base2d, gate)


def kernel(x, Wq, Wk, Wv, Wo, t_emb, W_mod, W_ff1, W_ff2):
    B, S, D = x.shape
    eps = 1e-5
    Dh = 96

    mod = t_emb @ W_mod
    sa, sha, ga, sm, shm, gm = jnp.split(mod, 6, axis=-1)

    def ln(h):
        m = h.mean(axis=-1, keepdims=True)
        v = h.var(axis=-1, keepdims=True)
        return (h - m) * lax.rsqrt(v + eps)

    x0 = x
    xa = ln(x0) * (1.0 + sa[:, None, :]) + sha[:, None, :]

    h_local = Wq.shape[1] // Dh
    Q = (xa @ Wq).reshape(B, S, h_local, Dh)
    K = (xa @ Wk).reshape(B, S, h_local, Dh)
    V = (xa @ Wv).reshape(B, S, h_local, Dh)
    scores = jnp.einsum("bihd,bjhd->bhij", Q, K) * (1.0 / (Dh ** 0.5))
    p = jax.nn.softmax(scores, axis=-1)
    o = jnp.einsum("bhij,bjhd->bihd", p, V).reshape(B, S, h_local * Dh)
    partial_attn = o @ Wo

    x1_2d = _ar_residual(
        partial_attn.reshape(B * S, D), x0.reshape(B * S, D), ga, 0
    )
    x1 = x1_2d.reshape(B, S, D)

    xm = ln(x1) * (1.0 + sm[:, None, :]) + shm[:, None, :]
    h = xm @ W_ff1
    h = h * jax.nn.sigmoid(h)
    partial_ff = h @ W_ff2

    out2d = _ar_residual(partial_ff.reshape(B * S, D), x1_2d, gm, 1)
    return out2d.reshape(B, S, D)


# device time: 107392 ns/iter; 2.4570x vs baseline; 1.0049x over previous
import jax
import jax.numpy as jnp
from jax import lax
from jax.experimental import pallas as pl
from jax.experimental.pallas import tpu as pltpu

N_DEV = 16
PLANE = 4
GROUP = 256
HALF = 128
SUB = 32


def _ar_residual(partial2d, base2d, gate, collective_id):
    rows, cols = partial2d.shape

    def body(p_ref, base_ref, gate_ref, out_ref,
             pbufA, pbufB, zbuf, sbufA, sbufB, zsbuf,
             cw_send, ccw_send, z_send, z2_send, dummy_sem,
             cwrs_recv, ccwrs_recv, zrs_recv, zag_recv,
             cwag_recv, ccwag_recv):
        my = lax.axis_index("i")
        z = my // PLANE
        k = my % PLANE
        right_p = z * PLANE + (k + 1) % PLANE
        left_p = z * PLANE + (k + 3) % PLANE
        up = ((z + 1) % PLANE) * PLANE + k
        down = ((z + 3) % PLANE) * PLANE + k

        A_k = (k + 1) % PLANE
        B_k = (k + 3) % PLANE

        def arow(g, j):
            return g * GROUP + j * SUB

        def brow(g, j):
            return g * GROUP + HALF + j * SUB

        def j_of(c):
            return (z - c) % PLANE

        class Link:

            def __init__(self, sems, dev):
                self.sems = sems
                self.dev = dev
                self.i = 0
                self.q = []

            def send(self, src, dst, recv_sem):
                rdma = pltpu.make_async_remote_copy(
                    src_ref=src, dst_ref=dst,
                    send_sem=self.sems.at[self.i % 4],
                    recv_sem=recv_sem,
                    device_id=(self.dev,),
                    device_id_type=pl.DeviceIdType.MESH,
                )
                rdma.start()
                self.i += 1
                self.q.append(rdma)
                if len(self.q) > 2:
                    self.q.pop(0).wait_send()

            def drain(self):
                for r in self.q:
                    r.wait_send()
                self.q = []

        cw = Link(cw_send, right_p)
        ccw = Link(ccw_send, left_p)
        zl = Link(z_send, up)
        zdl = Link(z2_send, down)

        def wait_recv(dst, sem_recv, dev):
            pltpu.make_async_remote_copy(
                src_ref=dst, dst_ref=dst, send_sem=dummy_sem.at[0],
                recv_sem=sem_recv, device_id=(dev,),
                device_id_type=pl.DeviceIdType.MESH,
            ).wait_recv()

        barrier_sem = pltpu.get_barrier_semaphore()
        for nbr in (left_p, right_p, up, down):
            pl.semaphore_signal(
                barrier_sem, inc=1,
                device_id=(nbr,), device_id_type=pl.DeviceIdType.MESH,
            )
        pl.semaphore_wait(barrier_sem, 4)

        for c in range(PLANE):
            j = j_of(c)
            ccw.send(p_ref.at[pl.ds(arow(k, j), SUB)],
                     pbufA.at[PLANE + c], cwrs_recv.at[1, c])
            cw.send(p_ref.at[pl.ds(arow((k + 3) % PLANE, j), SUB)],
                    pbufA.at[c], cwrs_recv.at[0, c])
            cw.send(p_ref.at[pl.ds(brow(k, j), SUB)],
                    pbufB.at[PLANE + c], ccwrs_recv.at[1, c])
            ccw.send(p_ref.at[pl.ds(brow((k + 1) % PLANE, j), SUB)],
                     pbufB.at[c], ccwrs_recv.at[0, c])
        for c in range(PLANE):
            j = j_of(c)
            wait_recv(pbufA.at[c], cwrs_recv.at[0, c], left_p)
            sbufA[c] = (
                p_ref[pl.ds(arow((k + 2) % PLANE, j), SUB), :] + pbufA[c]
            )
            cw.send(sbufA.at[c], pbufA.at[2 * PLANE + c],
                    cwrs_recv.at[2, c])
            wait_recv(pbufB.at[c], ccwrs_recv.at[0, c], right_p)
            sbufB[c] = (
                p_ref[pl.ds(brow((k + 2) % PLANE, j), SUB), :] + pbufB[c]
            )
            ccw.send(sbufB.at[c], pbufB.at[2 * PLANE + c],
                     ccwrs_recv.at[2, c])

        for c in range(PLANE):
            j = j_of(c)
            wait_recv(pbufA.at[PLANE + c], cwrs_recv.at[1, c], right_p)
            wait_recv(pbufA.at[2 * PLANE + c], cwrs_recv.at[2, c], left_p)
            wait_recv(pbufB.at[PLANE + c], ccwrs_recv.at[1, c], left_p)
            wait_recv(pbufB.at[2 * PLANE + c], ccwrs_recv.at[2, c], right_p)
            pa = (p_ref[pl.ds(arow(A_k, j), SUB), :]
                  + pbufA[PLANE + c] + pbufA[2 * PLANE + c])
            pb = (p_ref[pl.ds(brow(B_k, j), SUB), :]
                  + pbufB[PLANE + c] + pbufB[2 * PLANE + c])
            if c == 0:
                zsbuf[0, 0] = pa
                zdl.send(zsbuf.at[0, 0], zbuf.at[1, 0], zrs_recv.at[1, 0])
            elif c == 1:
                zsbuf[1, 0] = pa
                zl.send(zsbuf.at[1, 0], zbuf.at[0, 0], zrs_recv.at[0, 0])
                zsbuf[1, 1] = pb
                zdl.send(zsbuf.at[1, 1], zbuf.at[0, 1], zrs_recv.at[0, 1])
            elif c == 2:
                wait_recv(zbuf.at[0, 0], zrs_recv.at[0, 0], down)
                zsbuf[2, 0] = pa + zbuf[0, 0]
                zl.send(zsbuf.at[2, 0], zbuf.at[2, 0], zrs_recv.at[2, 0])
                zsbuf[2, 1] = pb
                zl.send(zsbuf.at[2, 1], zbuf.at[1, 1], zrs_recv.at[1, 1])
                wait_recv(zbuf.at[0, 1], zrs_recv.at[0, 1], up)
                zsbuf[0, 1] = (
                    p_ref[pl.ds(brow(B_k, z), SUB), :]
                    + pbufB[PLANE + 0] + pbufB[2 * PLANE + 0] + zbuf[0, 1]
                )
                zdl.send(zsbuf.at[0, 1], zbuf.at[2, 1], zrs_recv.at[2, 1])
            else:
                wait_recv(zbuf.at[1, 0], zrs_recv.at[1, 0], up)
                wait_recv(zbuf.at[2, 0], zrs_recv.at[2, 0], down)
                wait_recv(zbuf.at[1, 1], zrs_recv.at[1, 1], down)
                wait_recv(zbuf.at[2, 1], zrs_recv.at[2, 1], up)
                va = pa + zbuf[1, 0] + zbuf[2, 0]
                vb = pb + zbuf[1, 1] + zbuf[2, 1]
                for off, val in ((arow(A_k, j), va), (brow(B_k, j), vb)):
                    g_row = gate_ref[pl.ds(off // 512, 1), :]
                    out_ref[pl.ds(off, SUB), :] = (
                        base_ref[pl.ds(off, SUB), :] + g_row * val
                    )
        j_own = (z + 1) % PLANE

        def jag(r):
            return (z + 1 + (0, 3, 1, 2)[r]) % PLANE

        def out_at(rowoff):
            return out_ref.at[pl.ds(rowoff, SUB)]

        def plane_own_sends(r):
            a = out_at(arow(A_k, jag(r)))
            b = out_at(brow(B_k, jag(r)))
            cw.send(a, a, cwag_recv.at[0, r])
            ccw.send(a, a, cwag_recv.at[2, r])
            ccw.send(b, b, ccwag_recv.at[0, r])
            cw.send(b, b, ccwag_recv.at[2, r])

        own_a = out_at(arow(A_k, jag(0)))
        own_b = out_at(brow(B_k, jag(0)))
        zl.send(own_a, own_a, zag_recv.at[0, 0])
        zdl.send(own_a, own_a, zag_recv.at[1, 0])
        zl.send(own_b, own_b, zag_recv.at[0, 1])
        zdl.send(own_b, own_b, zag_recv.at[1, 1])
        plane_own_sends(0)
        wait_recv(out_at(arow(A_k, jag(1))), zag_recv.at[0, 0], down)
        wait_recv(out_at(brow(B_k, jag(1))), zag_recv.at[0, 1], down)
        zl.send(out_at(arow(A_k, jag(1))), out_at(arow(A_k, jag(1))),
                zag_recv.at[2, 0])
        plane_own_sends(1)
        wait_recv(out_at(arow(A_k, jag(2))), zag_recv.at[1, 0], up)
        wait_recv(out_at(brow(B_k, jag(2))), zag_recv.at[1, 1], up)
        zdl.send(out_at(brow(B_k, jag(2))), out_at(brow(B_k, jag(2))),
                 zag_recv.at[2, 1])
        plane_own_sends(2)
        wait_recv(out_at(arow(A_k, jag(3))), zag_recv.at[2, 0], down)
        wait_recv(out_at(brow(B_k, jag(3))), zag_recv.at[2, 1], up)
        plane_own_sends(3)
        for r in range(PLANE):
            wait_recv(out_at(arow(k, jag(r))), cwag_recv.at[0, r], left_p)
            cw.send(out_at(arow(k, jag(r))), out_at(arow(k, jag(r))),
                    cwag_recv.at[1, r])
            wait_recv(out_at(brow(k, jag(r))), ccwag_recv.at[0, r], right_p)
            ccw.send(out_at(brow(k, jag(r))), out_at(brow(k, jag(r))),
                     ccwag_recv.at[1, r])
        for r in range(PLANE):
            wait_recv(out_at(arow((k + 3) % PLANE, jag(r))),
                      cwag_recv.at[1, r], left_p)
            wait_recv(out_at(arow((k + 2) % PLANE, jag(r))),
                      cwag_recv.at[2, r], right_p)
            wait_recv(out_at(brow((k + 1) % PLANE, jag(r))),
                      ccwag_recv.at[1, r], right_p)
            wait_recv(out_at(brow((k + 2) % PLANE, jag(r))),
                      ccwag_recv.at[2, r], left_p)

        cw.drain()
        ccw.drain()
        zl.drain()
        zdl.drain()

    n_steps = PLANE - 1
    sem4 = pltpu.SemaphoreType.DMA((n_steps, PLANE))
    sem2 = pltpu.SemaphoreType.DMA((n_steps, 2))
    return pl.pallas_call(
        body,
        out_shape=jax.ShapeDtypeStruct((rows, cols), partial2d.dtype),
        in_specs=[
            pl.BlockSpec(memory_space=pltpu.VMEM),
            pl.BlockSpec(memory_space=pltpu.VMEM),
            pl.BlockSpec(memory_space=pltpu.VMEM),
        ],
        out_specs=pl.BlockSpec(memory_space=pltpu.VMEM),
        scratch_shapes=[
            pltpu.VMEM((n_steps * PLANE, SUB, cols), partial2d.dtype),
            pltpu.VMEM((n_steps * PLANE, SUB, cols), partial2d.dtype),
            pltpu.VMEM((n_steps, 2, SUB, cols), partial2d.dtype),
            pltpu.VMEM((2 * PLANE, SUB, cols), partial2d.dtype),
            pltpu.VMEM((2 * PLANE, SUB, cols), partial2d.dtype),
            pltpu.VMEM((n_steps, 2, SUB, cols), partial2d.dtype),
            pltpu.SemaphoreType.DMA((4,)),
            pltpu.SemaphoreType.DMA((4,)),
            pltpu.SemaphoreType.DMA((4,)),
            pltpu.SemaphoreType.DMA((4,)),
            pltpu.SemaphoreType.DMA((1,)),
            sem4,
            sem4,
            sem2,
            sem2,
            sem4,
            sem4,
        ],
        compiler_params=pltpu.CompilerParams(collective_id=collective_id),
    )(partial2d, base2d, gate)


def kernel(x, Wq, Wk, Wv, Wo, t_emb, W_mod, W_ff1, W_ff2):
    B, S, D = x.shape
    eps = 1e-5
    Dh = 96

    mod = t_emb @ W_mod
    sa, sha, ga, sm, shm, gm = jnp.split(mod, 6, axis=-1)

    def ln(h):
        m = h.mean(axis=-1, keepdims=True)
        v = h.var(axis=-1, keepdims=True)
        return (h - m) * lax.rsqrt(v + eps)

    x0 = x
    xa = ln(x0) * (1.0 + sa[:, None, :]) + sha[:, None, :]

    h_local = Wq.shape[1] // Dh
    Q = (xa @ Wq).reshape(B, S, h_local, Dh)
    K = (xa @ Wk).reshape(B, S, h_local, Dh)
    V = (xa @ Wv).reshape(B, S, h_local, Dh)
    scores = jnp.einsum("bihd,bjhd->bhij", Q, K) * (1.0 / (Dh ** 0.5))
    p = jax.nn.softmax(scores, axis=-1)
    o = jnp.einsum("bhij,bjhd->bihd", p, V).reshape(B, S, h_local * Dh)
    partial_attn = o @ Wo

    x1_2d = _ar_residual(
        partial_attn.reshape(B * S, D), x0.reshape(B * S, D), ga, 0
    )
    x1 = x1_2d.reshape(B, S, D)

    xm = ln(x1) * (1.0 + sm[:, None, :]) + shm[:, None, :]
    h = xm @ W_ff1
    h = h * jax.nn.sigmoid(h)
    partial_ff = h @ W_ff2

    out2d = _ar_residual(partial_ff.reshape(B * S, D), x1_2d, gm, 1)
    return out2d.reshape(B, S, D)
